# baseline (speedup 1.0000x reference)
import functools

import jax
import jax.numpy as jnp
from jax import lax
from jax.experimental import pallas as pl
from jax.experimental.pallas import tpu as pltpu
from jax.experimental.pallas import tpu_sc as plsc

N = 10000
E = 320000
C = 128
BN_EPS = 1e-5

NC = 2
NS = 16
NW = NC * NS
K = 80
NCHUNK = 128
EPT = K * NCHUNK
E_PAD = NW * EPT
R_PAD = 10112
ROWS_PER_TILE = R_PAD // NS
EW = 2 * K
SB = 4
NSB = NCHUNK // SB
EA_SCALE = float(1 << 20)


def _bn(x, g, b):
    m = jnp.mean(x, axis=0)
    v = jnp.mean((x - m) * (x - m), axis=0)
    return (x - m) * jax.lax.rsqrt(v + BN_EPS) * g + b


def _bn_in_kernel(x_ref, g_ref, b_ref, o_ref):
    o_ref[...] = _bn(x_ref[...], g_ref[...], b_ref[...])


def _dense_kernel(x_ref, p_ref, w_ref, b_ref, g_ref, bb_ref, o_ref):
    x = x_ref[...]
    h = x + p_ref[0, :N, :] + p_ref[1, :N, :]
    y = jnp.tanh(
        lax.dot_general(h, w_ref[...], (((1,), (1,)), ((), ())),
                        preferred_element_type=jnp.float32)
        + b_ref[...])
    o_ref[...] = _bn(y, g_ref[...], bb_ref[...])


def _final_kernel(x1_ref, q_ref, w_ref, b_ref, g_ref, bb_ref, fc_ref, o_ref):
    x1 = x1_ref[...]
    h = x1 + q_ref[0, :N, :] + q_ref[1, :N, :]
    y = jnp.tanh(
        lax.dot_general(h, w_ref[...], (((1,), (1,)), ((), ())),
                        preferred_element_type=jnp.float32)
        + b_ref[...])
    x2 = _bn(y, g_ref[...], bb_ref[...])
    x3 = jnp.tanh(
        lax.dot_general(x2, fc_ref[...], (((1,), (1,)), ((), ())),
                        preferred_element_type=jnp.float32))
    o_ref[:, 0:C] = x1
    o_ref[:, C:2 * C] = x2
    o_ref[:, 2 * C:3 * C] = x3


NBUF = 4


def _msg_pass(x, ebd, w, be):
    @functools.partial(
        pl.kernel,
        out_type=jax.ShapeDtypeStruct((NC, R_PAD, C), jnp.float32),
        mesh=plsc.VectorSubcoreMesh(core_axis_name="c", subcore_axis_name="s"),
        scratch_types=[
            pltpu.VMEM((K, C), jnp.float32),
            pltpu.VMEM((K, C), jnp.float32),
            pltpu.VMEM((K, C), jnp.float32),
            pltpu.VMEM((K, C), jnp.float32),
            pltpu.VMEM((2, SB * EW), jnp.int32),
            pltpu.VMEM((NBUF, K), jnp.int32),
            pltpu.VMEM((NBUF, K), jnp.int32),
            pltpu.VMEM((C,), jnp.float32),
            pltpu.VMEM((C,), jnp.float32),
            pltpu.VMEM_SHARED((R_PAD, C), jnp.float32),
            pltpu.SemaphoreType.DMA((NBUF,)),
            pltpu.SemaphoreType.DMA((NBUF,)),
            pltpu.SemaphoreType.DMA((2,)),
        ],
    )
    def k(x_hbm, ebd_hbm, w_hbm, be_hbm, out_hbm,
          rb0, rb1, rb2, rb3, ering, sidx, didx, w_v, be_v, aggr_s,
          gsem, ssem, esem):
        rows = [rb0, rb1, rb2, rb3]
        cid = lax.axis_index("c")
        sid = lax.axis_index("s")
        wid = sid * NC + cid

        pltpu.sync_copy(w_hbm, w_v)
        pltpu.sync_copy(be_hbm, be_v)
        ws = [w_v[pl.ds(i * 16, 16)] for i in range(8)]
        bs = [be_v[pl.ds(i * 16, 16)] for i in range(8)]

        def eissue(sb, slot):
            pltpu.async_copy(ebd_hbm.at[wid, sb], ering.at[slot],
                             esem.at[slot])

        def ewait(slot):
            pltpu.make_async_copy(
                ebd_hbm.at[wid, 0], ering.at[slot], esem.at[slot]).wait()

        def unpack_idx(slot, ci, p):
            base = ci * EW
            for g in range(K // 16):
                word = ering[slot, pl.ds(base + g * 16, 16)]
                sidx[p, pl.ds(g * 16, 16)] = jnp.bitwise_and(word, 0xFFFF)
                didx[p, pl.ds(g * 16, 16)] = lax.shift_right_logical(word, 16)

        def gissue(p):
            pltpu.async_copy(x_hbm.at[sidx.at[p]], rows[p], gsem.at[p])

        def gwait(p):
            pltpu.make_async_copy(
                x_hbm.at[pl.ds(0, K)], rows[p], gsem.at[p]).wait()

        def sissue(p):
            pltpu.async_copy(rows[p], aggr_s.at[didx.at[p]], ssem.at[p],
                             add=True)

        def swait(p):
            pltpu.make_async_copy(
                rows[p], aggr_s.at[pl.ds(0, K)], ssem.at[p]).wait()

        def zrow(i, carry):
            for s in range(8):
                rb0[i, pl.ds(s * 16, 16)] = jnp.zeros((16,), jnp.float32)
            return carry
        lax.fori_loop(0, K, zrow, 0)
        zfull = ROWS_PER_TILE // K
        for zc in range(zfull):
            base = sid * ROWS_PER_TILE + zc * K
            pltpu.sync_copy(rb0, aggr_s.at[pl.ds(base, K)])
        ztail = ROWS_PER_TILE - zfull * K
        if ztail:
            base = sid * ROWS_PER_TILE + zfull * K
            pltpu.sync_copy(rb0.at[pl.ds(0, ztail)],
                            aggr_s.at[pl.ds(base, ztail)])

        eissue(0, 0)
        eissue(1, 1)
        ewait(0)
        unpack_idx(0, 0, 0)
        unpack_idx(0, 1, 1)
        gissue(0)
        gissue(1)
        plsc.subcore_barrier()

        def compute(slot, ci, p):
            buf = rows[p]
            for g in range(K // 16):
                afix = ering[slot, pl.ds(ci * EW + K + g * 16, 16)]
                a16 = afix.astype(jnp.float32) * (1.0 / EA_SCALE)
                for t in range(16):
                    a_b = lax.gather(
                        a16, jnp.full((16, 1), t, jnp.int32),
                        dimension_numbers=lax.GatherDimensionNumbers(
                            offset_dims=(), collapsed_slice_dims=(0,),
                            start_index_map=(0,)),
                        slice_sizes=(1,),
                        mode=lax.GatherScatterMode.PROMISE_IN_BOUNDS)
                    e = g * 16 + t
                    for s in range(8):
                        v = buf[e, pl.ds(s * 16, 16)]
                        buf[e, pl.ds(s * 16, 16)] = jnp.maximum(
                            v + (a_b * ws[s] + bs[s]), 0.0)

        def body(i, carry):
            slot_cur = lax.rem(i, 2)
            slot_nxt = lax.rem(i + 1, 2)
            for p in range(NBUF):
                pn = (p + 2) % NBUF
                gwait(p)
                if p < 2:
                    @pl.when(i >= 1)
                    def _():
                        swait(pn)
                else:
                    swait(pn)
                if p == 2:
                    ewait(slot_nxt)
                unpack_idx(slot_cur if p < 2 else slot_nxt, (p + 2) % SB, pn)
                gissue(pn)
                compute(slot_cur, p, p)
                if p == 3:
                    eissue(lax.rem(i + 2, NSB), slot_cur)
                sissue(p)
            return carry
        lax.fori_loop(0, NSB, body, 0)
        gwait(0)
        gwait(1)
        ewait(1)
        swait(2)
        swait(3)
        plsc.subcore_barrier()

        pltpu.sync_copy(
            aggr_s.at[pl.ds(sid * ROWS_PER_TILE, ROWS_PER_TILE)],
            out_hbm.at[cid, pl.ds(sid * ROWS_PER_TILE, ROWS_PER_TILE)])

    return k(x, ebd, w, be)


def _prep_edges(edge_index, edge_attr):
    pad = E_PAD - E
    src = jnp.concatenate([edge_index[0], jnp.zeros((pad,), jnp.int32)])
    dst = jnp.concatenate([edge_index[1], jnp.full((pad,), N, jnp.int32)])
    ea = jnp.concatenate([edge_attr[:, 0], jnp.zeros((pad,), jnp.float32)])
    pk = jnp.bitwise_or(lax.shift_left(dst, 16), src)
    ea_fix = jnp.round(ea * EA_SCALE).astype(jnp.int32)
    ebd = jnp.concatenate([pk.reshape(NW, NCHUNK, K),
                           ea_fix.reshape(NW, NCHUNK, K)], axis=2)
    return ebd.reshape(NW, NSB, SB * EW)


def kernel(X, edge_index, edge_attr, bn_in_g, bn_in_b, We0, be0, W0, b0,
           bn0_g, bn0_b, We1, be1, W1, b1, bn1_g, bn1_b, fc1_W):
    ebd = _prep_edges(edge_index, edge_attr)

    f32 = jnp.float32
    x = pl.pallas_call(
        _bn_in_kernel,
        out_shape=jax.ShapeDtypeStruct((N, C), f32),
    )(X, bn_in_g.reshape(1, C), bn_in_b.reshape(1, C))

    p = _msg_pass(x, ebd, We0[:, 0], be0)
    x1 = pl.pallas_call(
        _dense_kernel,
        out_shape=jax.ShapeDtypeStruct((N, C), f32),
    )(x, p, W0, b0.reshape(1, C), bn0_g.reshape(1, C), bn0_b.reshape(1, C))

    q = _msg_pass(x1, ebd, We1[:, 0], be1)
    out = pl.pallas_call(
        _final_kernel,
        out_shape=jax.ShapeDtypeStruct((N, 3 * C), f32),
    )(x1, q, W1, b1.reshape(1, C), bn1_g.reshape(1, C), bn1_b.reshape(1, C),
      fc1_W)
    return out


# speedup vs baseline: 1.1298x; 1.1298x over previous
import functools

import jax
import jax.numpy as jnp
from jax import lax
from jax.experimental import pallas as pl
from jax.experimental.pallas import tpu as pltpu
from jax.experimental.pallas import tpu_sc as plsc

N = 10000
E = 320000
C = 128
BN_EPS = 1e-5

NC = 2
NS = 16
NW = NC * NS
K = 80
NCHUNK = 128
EPT = K * NCHUNK
E_PAD = NW * EPT
R_PAD = 10112
ROWS_PER_TILE = R_PAD // NS
NEB = 8
EW = 2 * K
EA_SCALE = float(1 << 20)


def _bn(x, g, b):
    m = jnp.mean(x, axis=0)
    v = jnp.mean((x - m) * (x - m), axis=0)
    return (x - m) * jax.lax.rsqrt(v + BN_EPS) * g + b


def _bn_in_kernel(x_ref, g_ref, b_ref, o_ref):
    o_ref[...] = _bn(x_ref[...], g_ref[...], b_ref[...])


def _dense_kernel(x_ref, p_ref, w_ref, b_ref, g_ref, bb_ref, o_ref):
    x = x_ref[...]
    h = x + p_ref[0, :N, :] + p_ref[1, :N, :]
    y = jnp.tanh(
        lax.dot_general(h, w_ref[...], (((1,), (1,)), ((), ())),
                        preferred_element_type=jnp.float32)
        + b_ref[...])
    o_ref[...] = _bn(y, g_ref[...], bb_ref[...])


def _final_kernel(x1_ref, q_ref, w_ref, b_ref, g_ref, bb_ref, fc_ref, o_ref):
    x1 = x1_ref[...]
    h = x1 + q_ref[0, :N, :] + q_ref[1, :N, :]
    y = jnp.tanh(
        lax.dot_general(h, w_ref[...], (((1,), (1,)), ((), ())),
                        preferred_element_type=jnp.float32)
        + b_ref[...])
    x2 = _bn(y, g_ref[...], bb_ref[...])
    x3 = jnp.tanh(
        lax.dot_general(x2, fc_ref[...], (((1,), (1,)), ((), ())),
                        preferred_element_type=jnp.float32))
    o_ref[:, 0:C] = x1
    o_ref[:, C:2 * C] = x2
    o_ref[:, 2 * C:3 * C] = x3


NBUF = 4


def _msg_pass(x, ebd, w, be):
    @functools.partial(
        pl.kernel,
        out_type=jax.ShapeDtypeStruct((NC, R_PAD, C), jnp.float32),
        mesh=plsc.VectorSubcoreMesh(core_axis_name="c", subcore_axis_name="s"),
        scratch_types=[
            pltpu.VMEM((K, C), jnp.float32),
            pltpu.VMEM((K, C), jnp.float32),
            pltpu.VMEM((K, C), jnp.float32),
            pltpu.VMEM((K, C), jnp.float32),
            pltpu.VMEM((NEB, EW), jnp.int32),
            pltpu.VMEM((NBUF, K), jnp.int32),
            pltpu.VMEM((NBUF, K), jnp.int32),
            pltpu.VMEM((C,), jnp.float32),
            pltpu.VMEM((C,), jnp.float32),
            pltpu.VMEM_SHARED((R_PAD, C), jnp.float32),
            pltpu.SemaphoreType.DMA((NBUF,)),
            pltpu.SemaphoreType.DMA((NBUF,)),
            pltpu.SemaphoreType.DMA((NEB,)),
        ],
    )
    def k(x_hbm, ebd_hbm, w_hbm, be_hbm, out_hbm,
          rb0, rb1, rb2, rb3, ering, sidx, didx, w_v, be_v, aggr_s,
          gsem, ssem, esem):
        rows = [rb0, rb1, rb2, rb3]
        cid = lax.axis_index("c")
        sid = lax.axis_index("s")
        wid = sid * NC + cid

        pltpu.sync_copy(w_hbm, w_v)
        pltpu.sync_copy(be_hbm, be_v)
        ws = [w_v[pl.ds(i * 16, 16)] for i in range(8)]
        bs = [be_v[pl.ds(i * 16, 16)] for i in range(8)]

        def eissue(j, q):
            pltpu.async_copy(ebd_hbm.at[wid, j], ering.at[q], esem.at[q])

        def ewait(q):
            pltpu.make_async_copy(
                ebd_hbm.at[wid, 0], ering.at[q], esem.at[q]).wait()

        def unpack_idx(q, p):
            for g in range(K // 16):
                word = ering[q, pl.ds(g * 16, 16)]
                sidx[p, pl.ds(g * 16, 16)] = jnp.bitwise_and(word, 0xFFFF)
                didx[p, pl.ds(g * 16, 16)] = lax.shift_right_logical(word, 16)

        def gissue(p):
            pltpu.async_copy(x_hbm.at[sidx.at[p]], rows[p], gsem.at[p])

        def gwait(p):
            pltpu.make_async_copy(
                x_hbm.at[pl.ds(0, K)], rows[p], gsem.at[p]).wait()

        def sissue(p):
            pltpu.async_copy(rows[p], aggr_s.at[didx.at[p]], ssem.at[p],
                             add=True)

        def swait(p):
            pltpu.make_async_copy(
                rows[p], aggr_s.at[pl.ds(0, K)], ssem.at[p]).wait()

        def zrow(i, carry):
            for s in range(8):
                rb0[i, pl.ds(s * 16, 16)] = jnp.zeros((16,), jnp.float32)
            return carry
        lax.fori_loop(0, K, zrow, 0)
        zfull = ROWS_PER_TILE // K
        for zc in range(zfull):
            base = sid * ROWS_PER_TILE + zc * K
            pltpu.sync_copy(rb0, aggr_s.at[pl.ds(base, K)])
        ztail = ROWS_PER_TILE - zfull * K
        if ztail:
            base = sid * ROWS_PER_TILE + zfull * K
            pltpu.sync_copy(rb0.at[pl.ds(0, ztail)],
                            aggr_s.at[pl.ds(base, ztail)])

        for q in range(4):
            eissue(q, q)
        ewait(0)
        ewait(1)
        unpack_idx(0, 0)
        unpack_idx(1, 1)
        gissue(0)
        gissue(1)
        plsc.subcore_barrier()

        def compute(q, p):
            buf = rows[p]
            for g in range(K // 16):
                afix = ering[q, pl.ds(K + g * 16, 16)]
                a16 = afix.astype(jnp.float32) * (1.0 / EA_SCALE)
                for t in range(16):
                    a_b = lax.gather(
                        a16, jnp.full((16, 1), t, jnp.int32),
                        dimension_numbers=lax.GatherDimensionNumbers(
                            offset_dims=(), collapsed_slice_dims=(0,),
                            start_index_map=(0,)),
                        slice_sizes=(1,),
                        mode=lax.GatherScatterMode.PROMISE_IN_BOUNDS)
                    e = g * 16 + t
                    for s in range(8):
                        v = buf[e, pl.ds(s * 16, 16)]
                        buf[e, pl.ds(s * 16, 16)] = jnp.maximum(
                            v + (a_b * ws[s] + bs[s]), 0.0)

        def body(i, carry):
            for p in range(NBUF):
                j = i * NBUF + p
                pn = (p + 2) % NBUF
                q = lax.rem(j, NEB)
                qn2 = lax.rem(j + 2, NEB)
                qn4 = lax.rem(j + 4, NEB)
                gwait(p)
                if p < 2:
                    @pl.when(i >= 1)
                    def _():
                        swait(pn)
                else:
                    swait(pn)
                ewait(qn2)
                unpack_idx(qn2, pn)
                gissue(pn)
                eissue(lax.rem(j + 4, NCHUNK), qn4)
                compute(q, p)
                sissue(p)
            return carry
        lax.fori_loop(0, NCHUNK // NBUF, body, 0)
        gwait(0)
        gwait(1)
        ewait(2)
        ewait(3)
        swait(2)
        swait(3)
        plsc.subcore_barrier()

        pltpu.sync_copy(
            aggr_s.at[pl.ds(sid * ROWS_PER_TILE, ROWS_PER_TILE)],
            out_hbm.at[cid, pl.ds(sid * ROWS_PER_TILE, ROWS_PER_TILE)])

    return k(x, ebd, w, be)


def _prep_edges(edge_index, edge_attr):
    pad = E_PAD - E
    src = jnp.concatenate([edge_index[0], jnp.zeros((pad,), jnp.int32)])
    dst = jnp.concatenate([edge_index[1], jnp.full((pad,), N, jnp.int32)])
    ea = jnp.concatenate([edge_attr[:, 0], jnp.zeros((pad,), jnp.float32)])
    pk = jnp.bitwise_or(lax.shift_left(dst, 16), src)
    ea_fix = jnp.round(ea * EA_SCALE).astype(jnp.int32)
    return jnp.concatenate([pk.reshape(NW, NCHUNK, K),
                            ea_fix.reshape(NW, NCHUNK, K)], axis=2)


def kernel(X, edge_index, edge_attr, bn_in_g, bn_in_b, We0, be0, W0, b0,
           bn0_g, bn0_b, We1, be1, W1, b1, bn1_g, bn1_b, fc1_W):
    ebd = _prep_edges(edge_index, edge_attr)

    f32 = jnp.float32
    x = pl.pallas_call(
        _bn_in_kernel,
        out_shape=jax.ShapeDtypeStruct((N, C), f32),
    )(X, bn_in_g.reshape(1, C), bn_in_b.reshape(1, C))

    p = _msg_pass(x, ebd, We0[:, 0], be0)
    x1 = pl.pallas_call(
        _dense_kernel,
        out_shape=jax.ShapeDtypeStruct((N, C), f32),
    )(x, p, W0, b0.reshape(1, C), bn0_g.reshape(1, C), bn0_b.reshape(1, C))

    q = _msg_pass(x1, ebd, We1[:, 0], be1)
    out = pl.pallas_call(
        _final_kernel,
        out_shape=jax.ShapeDtypeStruct((N, 3 * C), f32),
    )(x1, q, W1, b1.reshape(1, C), bn1_g.reshape(1, C), bn1_b.reshape(1, C),
      fc1_W)
    return out
